# Initial kernel scaffold; baseline (speedup 1.0000x reference)
import jax
import jax.numpy as jnp
from jax.experimental import pallas as pl


def kernel(inputs, targets):
    raise NotImplementedError("write your pallas kernel here")


# speedup vs baseline: 2.1628x; 2.1628x over previous
import jax
import jax.numpy as jnp
from jax.experimental import pallas as pl
from jax.experimental.pallas import tpu as pltpu

_K = 20
_ROWS = 128
_COLS = 32768
_BLOCK_ROWS = 16

_NEG = -1e30
_POS = 1e30


def _bce(x, t):
    return jnp.maximum(x, 0.0) - x * t + jax.nn.softplus(-jnp.abs(x))


def _topk_sum_rows(bce):
    r = bce.shape[0]
    cur = jnp.full((r, 1), _POS, dtype=jnp.float32)
    tk = jnp.full((r, 1), _NEG, dtype=jnp.float32)
    found = jnp.zeros((r, 1), dtype=jnp.bool_)

    for _ in range(_K):
        masked = jnp.where(bce < cur, bce, _NEG)
        nxt = jnp.max(masked, axis=1, keepdims=True)
        cnt = jnp.sum((bce >= nxt).astype(jnp.float32), axis=1, keepdims=True)
        newly = jnp.logical_and(cnt >= _K, jnp.logical_not(found))
        tk = jnp.where(newly, nxt, tk)
        found = jnp.logical_or(found, newly)
        cur = jnp.where(found, cur, nxt)
    gt = bce > tk
    sum_gt = jnp.sum(jnp.where(gt, bce, 0.0), axis=1, keepdims=True)
    cnt_gt = jnp.sum(gt.astype(jnp.float32), axis=1, keepdims=True)
    return sum_gt + (_K - cnt_gt) * tk


def _kernel_body(inp_ref, tgt_ref, out_ref):
    bce = _bce(inp_ref[...], tgt_ref[...])
    block_total = jnp.sum(_topk_sum_rows(bce))

    @pl.when(pl.program_id(0) == 0)
    def _():
        out_ref[...] = jnp.zeros((1, 1), dtype=jnp.float32)

    out_ref[...] += jnp.reshape(block_total, (1, 1))


def kernel(inputs, targets):
    grid = _ROWS // _BLOCK_ROWS
    total = pl.pallas_call(
        _kernel_body,
        grid=(grid,),
        in_specs=[
            pl.BlockSpec((_BLOCK_ROWS, _COLS), lambda i: (i, 0)),
            pl.BlockSpec((_BLOCK_ROWS, _COLS), lambda i: (i, 0)),
        ],
        out_specs=pl.BlockSpec((1, 1), lambda i: (0, 0)),
        out_shape=jax.ShapeDtypeStruct((1, 1), jnp.float32),
    )(inputs, targets)
    return total[0, 0] / (_ROWS * _K)
